# Initial kernel scaffold; baseline (speedup 1.0000x reference)
import jax
import jax.numpy as jnp
from jax.experimental import pallas as pl


def kernel(coords, box, pairs, q, p, t):
    raise NotImplementedError("write your pallas kernel here")


# speedup vs baseline: 9.8117x; 9.8117x over previous
import functools

import jax
import jax.numpy as jnp
from jax import lax
from jax.experimental import pallas as pl
from jax.experimental.pallas import tpu as pltpu
from jax.experimental.pallas import tpu_sc as plsc

_CUTOFF2 = 10000.0
_LANES = 16
_CHUNK = 128
_NC = 2
_NS = 16
_NW = _NC * _NS

_ROWS = [
    "r,-tx,-ty,-tz,txx,txy,txz,tyy,tyz,tzz",
    "tx,-txx,-txy,-txz,txxx,txxy,txxz,tyyx,txyz,tzzx",
    "ty,-txy,-tyy,-tyz,txxy,tyyx,txyz,tyyy,tyyz,tzzy",
    "tz,-txz,-tyz,-tzz,txxz,txyz,tzzx,tyyz,tzzy,tzzz",
    "txx,-txxx,-txxy,-txxz,txxxx,txxxy,txxxz,txxyy,txxyz,txxzz",
    "txy,-txxy,-tyyx,-txyz,txxxy,txxyy,txxyz,tyyyx,tyyxz,tzzxy",
    "txz,-txxz,-txyz,-tzzx,txxxz,txxyz,txxzz,tyyxz,tzzxy,tzzzx",
    "tyy,-tyyx,-tyyy,-tyyz,txxyy,tyyyx,tyyxz,tyyyy,tyyyz,tyyzz",
    "tyz,-txyz,-tyyz,-tzzy,txxyz,tyyxz,tzzxy,tyyyz,tyyzz,tzzzy",
    "tzz,-tzzx,-tzzy,-tzzz,txxzz,tzzxy,tzzzx,tyyzz,tzzzy,tzzzz",
]

_MAGIC = 12582912.0


def _bfrn(x):
    i = lax.bitcast_convert_type(x, jnp.uint32)
    odd = lax.shift_right_logical(i, jnp.uint32(16)) & jnp.uint32(1)
    i = (i + jnp.uint32(0x7FFF) + odd) & jnp.uint32(0xFFFF0000)
    return lax.bitcast_convert_type(i, jnp.float32)


def _rsqrt_newton(x):
    ib = lax.bitcast_convert_type(x, jnp.int32)
    ib = jnp.int32(0x5F3759DF) - lax.shift_right_logical(ib, 1)
    y = lax.bitcast_convert_type(ib, jnp.float32)
    for _ in range(3):
        y = y * (1.5 - 0.5 * x * y * y)
    return y


def _edge_energy(xi, yi, zi, xj, yj, zj, mi, mj, bi, bx):
    dx = _bfrn(xj - xi)
    dy = _bfrn(yj - yi)
    dz = _bfrn(zj - zi)
    s0 = dx * bi[0] + dy * bi[3] + dz * bi[6]
    s1 = dx * bi[1] + dy * bi[4] + dz * bi[7]
    s2 = dx * bi[2] + dy * bi[5] + dz * bi[8]
    s0 = _bfrn(s0 - ((s0 + _MAGIC) - _MAGIC))
    s1 = _bfrn(s1 - ((s1 + _MAGIC) - _MAGIC))
    s2 = _bfrn(s2 - ((s2 + _MAGIC) - _MAGIC))
    x = s0 * bx[0] + s1 * bx[3] + s2 * bx[6]
    y = s0 * bx[1] + s1 * bx[4] + s2 * bx[7]
    z = s0 * bx[2] + s1 * bx[5] + s2 * bx[8]
    dr2 = x * x + y * y + z * z
    drInv = _rsqrt_newton(dr2)

    drInv2 = drInv * drInv
    drInv3 = drInv2 * drInv
    drInv5 = drInv3 * drInv2
    drInv7 = drInv5 * drInv2
    drInv9 = drInv7 * drInv2
    x2, y2, z2 = x * x, y * y, z * z
    xy, xz, yz = x * y, x * z, y * z
    t = {}
    t["r"] = drInv
    t["tx"] = -x * drInv3
    t["ty"] = -y * drInv3
    t["tz"] = -z * drInv3
    t["txx"] = 3 * x2 * drInv5 - drInv3
    t["txy"] = 3 * xy * drInv5
    t["txz"] = 3 * xz * drInv5
    t["tyy"] = 3 * y2 * drInv5 - drInv3
    t["tyz"] = 3 * yz * drInv5
    t["tzz"] = 3 * z2 * drInv5 - drInv3
    t["txxx"] = -15 * x2 * x * drInv7 + 9 * x * drInv5
    t["txxy"] = -15 * x2 * y * drInv7 + 3 * y * drInv5
    t["txxz"] = -15 * x2 * z * drInv7 + 3 * z * drInv5
    t["tyyy"] = -15 * y2 * y * drInv7 + 9 * y * drInv5
    t["tyyx"] = -15 * y2 * x * drInv7 + 3 * x * drInv5
    t["tyyz"] = -15 * y2 * z * drInv7 + 3 * z * drInv5
    t["tzzz"] = -15 * z2 * z * drInv7 + 9 * z * drInv5
    t["tzzx"] = -15 * z2 * x * drInv7 + 3 * x * drInv5
    t["tzzy"] = -15 * z2 * y * drInv7 + 3 * y * drInv5
    t["txyz"] = -15 * x * y * z * drInv7
    t["txxxx"] = 105 * x2 * x2 * drInv9 - 90 * x2 * drInv7 + 9 * drInv5
    t["txxxy"] = 105 * x2 * xy * drInv9 - 45 * xy * drInv7
    t["txxxz"] = 105 * x2 * xz * drInv9 - 45 * xz * drInv7
    t["txxyy"] = 105 * x2 * y2 * drInv9 - 15 * (x2 + y2) * drInv7 + 3 * drInv5
    t["txxzz"] = 105 * x2 * z2 * drInv9 - 15 * (x2 + z2) * drInv7 + 3 * drInv5
    t["txxyz"] = 105 * x2 * yz * drInv9 - 15 * yz * drInv7
    t["tyyyy"] = 105 * y2 * y2 * drInv9 - 90 * y2 * drInv7 + 9 * drInv5
    t["tyyyx"] = 105 * y2 * xy * drInv9 - 45 * xy * drInv7
    t["tyyyz"] = 105 * y2 * yz * drInv9 - 45 * yz * drInv7
    t["tyyzz"] = 105 * y2 * z2 * drInv9 - 15 * (y2 + z2) * drInv7 + 3 * drInv5
    t["tyyxz"] = 105 * y2 * xz * drInv9 - 15 * xz * drInv7
    t["tzzzz"] = 105 * z2 * z2 * drInv9 - 90 * z2 * drInv7 + 9 * drInv5
    t["tzzzx"] = 105 * z2 * xz * drInv9 - 45 * xz * drInv7
    t["tzzzy"] = 105 * z2 * yz * drInv9 - 45 * yz * drInv7
    t["tzzxy"] = 105 * z2 * xy * drInv9 - 15 * xy * drInv7

    ene = None
    for a, row in enumerate(_ROWS):
        d = None
        for b, term in enumerate(row.split(",")):
            neg = term.startswith("-")
            prod = t[term.lstrip("-")] * mi[b]
            if d is None:
                d = -prod if neg else prod
            elif neg:
                d = d - prod
            else:
                d = d + prod
        contrib = mj[a] * d
        ene = contrib if ene is None else ene + contrib
    return ene, dr2


def _sc_body(n_chunks_w, n_edges, tab_hbm, src_hbm, dst_hbm, boxes_hbm,
             out_hbm, idx_s, idx_d, rows_s, rows_d, box_v, acc_v, sem):
    wid = lax.axis_index("s") * _NC + lax.axis_index("c")
    pltpu.sync_copy(boxes_hbm, box_v)
    bv0 = box_v[pl.ds(0, 16)]
    bv1 = box_v[pl.ds(16, 16)]
    bvals = [bv0[i] for i in range(16)] + [bv1[i] for i in range(16)]
    bi = bvals[0:9]
    bx = bvals[9:18]
    base_w = wid * (n_chunks_w * _CHUNK)
    lanes = lax.iota(jnp.int32, _LANES)

    def chunk_body(g, acc):
        base = base_w + g * _CHUNK
        pltpu.sync_copy(src_hbm.at[pl.ds(base, _CHUNK)], idx_s)
        pltpu.sync_copy(dst_hbm.at[pl.ds(base, _CHUNK)], idx_d)
        cp_s = pltpu.async_copy(tab_hbm.at[idx_s], rows_s, sem)
        cp_d = pltpu.async_copy(tab_hbm.at[idx_d], rows_d, sem)
        cp_s.wait()
        cp_d.wait()

        def step_body(s, acc2):
            rs = s * _LANES + lanes

            def col(ref, c):
                cv = jnp.full((_LANES,), c, jnp.int32)
                return plsc.load_gather(ref, [rs, cv])

            xi = col(rows_s, 0)
            yi = col(rows_s, 1)
            zi = col(rows_s, 2)
            xj = col(rows_d, 0)
            yj = col(rows_d, 1)
            zj = col(rows_d, 2)
            mi = [col(rows_s, 3 + b) for b in range(10)]
            mj = [col(rows_d, 3 + b) for b in range(10)]
            ene, dr2 = _edge_energy(xi, yi, zi, xj, yj, zj, mi, mj, bi, bx)
            eid = base + rs
            ok = (dr2 <= _CUTOFF2) & (eid < n_edges)
            return acc2 + jnp.where(ok, ene, 0.0)

        return lax.fori_loop(0, _CHUNK // _LANES, step_body, acc)

    acc = lax.fori_loop(0, n_chunks_w, chunk_body,
                        jnp.zeros((_LANES,), jnp.float32))
    acc_v[...] = acc
    pltpu.sync_copy(acc_v, out_hbm.at[wid])


@functools.partial(jax.jit, static_argnames=("n_chunks_w", "n_edges"))
def _sc_call(tab, src_p, dst_p, boxes, n_chunks_w, n_edges):
    mesh = plsc.VectorSubcoreMesh(core_axis_name="c", subcore_axis_name="s")
    grid_kernel = functools.partial(
        pl.kernel,
        mesh=mesh,
        compiler_params=pltpu.CompilerParams(
            needs_layout_passes=False, use_tc_tiling_on_sc=False),
        out_type=jax.ShapeDtypeStruct((_NW, _LANES), jnp.float32),
        scratch_types=[
            pltpu.VMEM((_CHUNK,), jnp.int32),
            pltpu.VMEM((_CHUNK,), jnp.int32),
            pltpu.VMEM((_CHUNK, _LANES), jnp.float32),
            pltpu.VMEM((_CHUNK, _LANES), jnp.float32),
            pltpu.VMEM((32,), jnp.float32),
            pltpu.VMEM((_LANES,), jnp.float32),
            pltpu.SemaphoreType.DMA,
        ],
    )
    body = functools.partial(_sc_body, n_chunks_w, n_edges)
    return grid_kernel(body)(tab, src_p, dst_p, boxes)


def kernel(coords, box, pairs, q, p, t):
    n = coords.shape[0]
    n_edges = pairs.shape[0]
    binv = jnp.linalg.inv(box)
    quad = jnp.stack([
        t[:, 0, 0] / 3,
        (t[:, 0, 1] + t[:, 1, 0]) / 3,
        (t[:, 0, 2] + t[:, 2, 0]) / 3,
        t[:, 1, 1] / 3,
        (t[:, 1, 2] + t[:, 2, 1]) / 3,
        t[:, 2, 2] / 3,
    ], axis=1)
    tab = jnp.concatenate([
        coords.astype(jnp.float32),
        q[:, None].astype(jnp.float32),
        p.astype(jnp.float32),
        quad.astype(jnp.float32),
        jnp.zeros((n, 3), jnp.float32),
    ], axis=1)
    per_w = -(-n_edges // (_NW * _CHUNK)) * _CHUNK
    n_chunks_w = per_w // _CHUNK
    e_pad = per_w * _NW
    pad = e_pad - n_edges
    src_p = jnp.concatenate(
        [pairs[:, 0], jnp.zeros((pad,), jnp.int32)]).astype(jnp.int32)
    dst_p = jnp.concatenate(
        [pairs[:, 1], jnp.ones((pad,), jnp.int32)]).astype(jnp.int32)
    boxes = jnp.concatenate([
        _bfrn(binv.reshape(-1).astype(jnp.float32)),
        _bfrn(box.reshape(-1).astype(jnp.float32)),
        jnp.zeros((14,), jnp.float32),
    ])
    out = _sc_call(tab, src_p, dst_p, boxes, n_chunks_w, n_edges)
    return jnp.sum(out)


# speedup vs baseline: 10.3176x; 1.0516x over previous
import functools

import jax
import jax.numpy as jnp
from jax import lax
from jax.experimental import pallas as pl
from jax.experimental.pallas import tpu as pltpu
from jax.experimental.pallas import tpu_sc as plsc

_CUTOFF2 = 10000.0
_LANES = 16
_CHUNK = 128
_NC = 2
_NS = 16
_NW = _NC * _NS

_ROWS = [
    "r,-tx,-ty,-tz,txx,txy,txz,tyy,tyz,tzz",
    "tx,-txx,-txy,-txz,txxx,txxy,txxz,tyyx,txyz,tzzx",
    "ty,-txy,-tyy,-tyz,txxy,tyyx,txyz,tyyy,tyyz,tzzy",
    "tz,-txz,-tyz,-tzz,txxz,txyz,tzzx,tyyz,tzzy,tzzz",
    "txx,-txxx,-txxy,-txxz,txxxx,txxxy,txxxz,txxyy,txxyz,txxzz",
    "txy,-txxy,-tyyx,-txyz,txxxy,txxyy,txxyz,tyyyx,tyyxz,tzzxy",
    "txz,-txxz,-txyz,-tzzx,txxxz,txxyz,txxzz,tyyxz,tzzxy,tzzzx",
    "tyy,-tyyx,-tyyy,-tyyz,txxyy,tyyyx,tyyxz,tyyyy,tyyyz,tyyzz",
    "tyz,-txyz,-tyyz,-tzzy,txxyz,tyyxz,tzzxy,tyyyz,tyyzz,tzzzy",
    "tzz,-tzzx,-tzzy,-tzzz,txxzz,tzzxy,tzzzx,tyyzz,tzzzy,tzzzz",
]

_MAGIC = 12582912.0


def _build_entries():
    by_name = {}
    order = []
    for a, row in enumerate(_ROWS):
        for b, term in enumerate(row.split(",")):
            sgn = -1 if term.startswith("-") else 1
            nm = term.lstrip("-")
            if nm not in by_name:
                by_name[nm] = []
                order.append(nm)
            by_name[nm].append((a, b, sgn))
    return [(nm, by_name[nm]) for nm in order]


_ENTRIES = _build_entries()


def _bfrn(x):
    i = lax.bitcast_convert_type(x, jnp.uint32)
    odd = lax.shift_right_logical(i, jnp.uint32(16)) & jnp.uint32(1)
    i = (i + jnp.uint32(0x7FFF) + odd) & jnp.uint32(0xFFFF0000)
    return lax.bitcast_convert_type(i, jnp.float32)


def _rsqrt_newton(x):
    ib = lax.bitcast_convert_type(x, jnp.int32)
    ib = jnp.int32(0x5F3759DF) - lax.shift_right_logical(ib, 1)
    y = lax.bitcast_convert_type(ib, jnp.float32)
    for _ in range(3):
        y = y * (1.5 - 0.5 * x * y * y)
    return y


def _edge_energy(xi, yi, zi, xj, yj, zj, mi, mj, bi, bx):
    dx = _bfrn(xj - xi)
    dy = _bfrn(yj - yi)
    dz = _bfrn(zj - zi)
    s0 = dx * bi[0] + dy * bi[3] + dz * bi[6]
    s1 = dx * bi[1] + dy * bi[4] + dz * bi[7]
    s2 = dx * bi[2] + dy * bi[5] + dz * bi[8]
    s0 = _bfrn(s0 - ((s0 + _MAGIC) - _MAGIC))
    s1 = _bfrn(s1 - ((s1 + _MAGIC) - _MAGIC))
    s2 = _bfrn(s2 - ((s2 + _MAGIC) - _MAGIC))
    x = s0 * bx[0] + s1 * bx[3] + s2 * bx[6]
    y = s0 * bx[1] + s1 * bx[4] + s2 * bx[7]
    z = s0 * bx[2] + s1 * bx[5] + s2 * bx[8]
    dr2 = x * x + y * y + z * z
    drInv = _rsqrt_newton(dr2)

    dri2 = drInv * drInv
    dri3 = dri2 * drInv
    dri5 = dri3 * dri2
    dri7 = dri5 * dri2
    dri9 = dri7 * dri2
    c5 = 3.0 * dri5
    c7 = 15.0 * dri7
    c9 = 105.0 * dri9
    c5_3 = 3.0 * c5
    c7_3 = 3.0 * c7
    nd3 = 0.0 - dri3
    nc7 = 0.0 - c7
    x2, y2, z2 = x * x, y * y, z * z
    xy, xz, yz = x * y, x * z, y * z
    u_x, u_y, u_z = x2 * c7, y2 * c7, z2 * c7
    g_x, g_y, g_z = x2 * c9, y2 * c9, z2 * c9
    tv = {
        "r": lambda: drInv,
        "tx": lambda: x * nd3,
        "ty": lambda: y * nd3,
        "tz": lambda: z * nd3,
        "txx": lambda: x2 * c5 - dri3,
        "txy": lambda: xy * c5,
        "txz": lambda: xz * c5,
        "tyy": lambda: y2 * c5 - dri3,
        "tyz": lambda: yz * c5,
        "tzz": lambda: z2 * c5 - dri3,
        "txxx": lambda: x * (c5_3 - u_x),
        "txxy": lambda: y * (c5 - u_x),
        "txxz": lambda: z * (c5 - u_x),
        "tyyy": lambda: y * (c5_3 - u_y),
        "tyyx": lambda: x * (c5 - u_y),
        "tyyz": lambda: z * (c5 - u_y),
        "tzzz": lambda: z * (c5_3 - u_z),
        "tzzx": lambda: x * (c5 - u_z),
        "tzzy": lambda: y * (c5 - u_z),
        "txyz": lambda: (xy * nc7) * z,
        "txxxx": lambda: x2 * g_x - 6.0 * u_x + c5_3,
        "txxxy": lambda: xy * (g_x - c7_3),
        "txxxz": lambda: xz * (g_x - c7_3),
        "txxyy": lambda: y2 * g_x - (u_x + u_y) + c5,
        "txxzz": lambda: z2 * g_x - (u_x + u_z) + c5,
        "txxyz": lambda: yz * (g_x - c7),
        "tyyyy": lambda: y2 * g_y - 6.0 * u_y + c5_3,
        "tyyyx": lambda: xy * (g_y - c7_3),
        "tyyyz": lambda: yz * (g_y - c7_3),
        "tyyzz": lambda: z2 * g_y - (u_y + u_z) + c5,
        "tyyxz": lambda: xz * (g_y - c7),
        "tzzzz": lambda: z2 * g_z - 6.0 * u_z + c5_3,
        "tzzzx": lambda: xz * (g_z - c7_3),
        "tzzzy": lambda: yz * (g_z - c7_3),
        "tzzxy": lambda: xy * (g_z - c7),
    }
    ene = None
    for name, entries in _ENTRIES:
        c = None
        for a, b, sgn in entries:
            prod = mj[a] * mi[b]
            if c is None:
                c = -prod if sgn < 0 else prod
            elif sgn < 0:
                c = c - prod
            else:
                c = c + prod
        contrib = tv[name]() * c
        ene = contrib if ene is None else ene + contrib
    return ene, dr2


def _sc_body(n_chunks_w, n_edges, tab_hbm, src_hbm, dst_hbm, boxes_hbm,
             out_hbm, idx_s, idx_d, rows_s, rows_d, box_v, acc_v, sem):
    wid = lax.axis_index("s") * _NC + lax.axis_index("c")
    pltpu.sync_copy(boxes_hbm, box_v)
    bv0 = box_v[pl.ds(0, 16)]
    bv1 = box_v[pl.ds(16, 16)]
    bvals = [bv0[i] for i in range(16)] + [bv1[i] for i in range(16)]
    bi = bvals[0:9]
    bx = bvals[9:18]
    base_w = wid * (n_chunks_w * _CHUNK)
    lanes = lax.iota(jnp.int32, _LANES)

    def chunk_body(g, acc):
        base = base_w + g * _CHUNK
        pltpu.sync_copy(src_hbm.at[pl.ds(base, _CHUNK)], idx_s)
        pltpu.sync_copy(dst_hbm.at[pl.ds(base, _CHUNK)], idx_d)
        cp_s = pltpu.async_copy(tab_hbm.at[idx_s], rows_s, sem)
        cp_d = pltpu.async_copy(tab_hbm.at[idx_d], rows_d, sem)
        cp_s.wait()
        cp_d.wait()

        def step_body(s, acc2):
            rs = s * _LANES + lanes

            def col(ref, c):
                cv = jnp.full((_LANES,), c, jnp.int32)
                return plsc.load_gather(ref, [rs, cv])

            xi = col(rows_s, 0)
            yi = col(rows_s, 1)
            zi = col(rows_s, 2)
            xj = col(rows_d, 0)
            yj = col(rows_d, 1)
            zj = col(rows_d, 2)
            mi = [col(rows_s, 3 + b) for b in range(10)]
            mj = [col(rows_d, 3 + b) for b in range(10)]
            ene, dr2 = _edge_energy(xi, yi, zi, xj, yj, zj, mi, mj, bi, bx)
            eid = base + rs
            ok = (dr2 <= _CUTOFF2) & (eid < n_edges)
            return acc2 + jnp.where(ok, ene, 0.0)

        return lax.fori_loop(0, _CHUNK // _LANES, step_body, acc)

    acc = lax.fori_loop(0, n_chunks_w, chunk_body,
                        jnp.zeros((_LANES,), jnp.float32))
    acc_v[...] = acc
    pltpu.sync_copy(acc_v, out_hbm.at[wid])


@functools.partial(jax.jit, static_argnames=("n_chunks_w", "n_edges"))
def _sc_call(tab, src_p, dst_p, boxes, n_chunks_w, n_edges):
    mesh = plsc.VectorSubcoreMesh(core_axis_name="c", subcore_axis_name="s")
    grid_kernel = functools.partial(
        pl.kernel,
        mesh=mesh,
        compiler_params=pltpu.CompilerParams(
            needs_layout_passes=False, use_tc_tiling_on_sc=False),
        out_type=jax.ShapeDtypeStruct((_NW, _LANES), jnp.float32),
        scratch_types=[
            pltpu.VMEM((_CHUNK,), jnp.int32),
            pltpu.VMEM((_CHUNK,), jnp.int32),
            pltpu.VMEM((_CHUNK, _LANES), jnp.float32),
            pltpu.VMEM((_CHUNK, _LANES), jnp.float32),
            pltpu.VMEM((32,), jnp.float32),
            pltpu.VMEM((_LANES,), jnp.float32),
            pltpu.SemaphoreType.DMA,
        ],
    )
    body = functools.partial(_sc_body, n_chunks_w, n_edges)
    return grid_kernel(body)(tab, src_p, dst_p, boxes)


def kernel(coords, box, pairs, q, p, t):
    n = coords.shape[0]
    n_edges = pairs.shape[0]
    binv = jnp.linalg.inv(box)
    quad = jnp.stack([
        t[:, 0, 0] / 3,
        (t[:, 0, 1] + t[:, 1, 0]) / 3,
        (t[:, 0, 2] + t[:, 2, 0]) / 3,
        t[:, 1, 1] / 3,
        (t[:, 1, 2] + t[:, 2, 1]) / 3,
        t[:, 2, 2] / 3,
    ], axis=1)
    tab = jnp.concatenate([
        coords.astype(jnp.float32),
        q[:, None].astype(jnp.float32),
        p.astype(jnp.float32),
        quad.astype(jnp.float32),
        jnp.zeros((n, 3), jnp.float32),
    ], axis=1)
    per_w = -(-n_edges // (_NW * _CHUNK)) * _CHUNK
    n_chunks_w = per_w // _CHUNK
    e_pad = per_w * _NW
    pad = e_pad - n_edges
    src_p = jnp.concatenate(
        [pairs[:, 0], jnp.zeros((pad,), jnp.int32)]).astype(jnp.int32)
    dst_p = jnp.concatenate(
        [pairs[:, 1], jnp.ones((pad,), jnp.int32)]).astype(jnp.int32)
    boxes = jnp.concatenate([
        _bfrn(binv.reshape(-1).astype(jnp.float32)),
        _bfrn(box.reshape(-1).astype(jnp.float32)),
        jnp.zeros((14,), jnp.float32),
    ])
    out = _sc_call(tab, src_p, dst_p, boxes, n_chunks_w, n_edges)
    return jnp.sum(out)
